# baseline (speedup 1.0000x reference)
import functools

import jax
import jax.numpy as jnp
from jax import lax
from jax.experimental import pallas as pl
from jax.experimental.pallas import tpu as pltpu
from jax.experimental.pallas import tpu_sc as plsc

B, D, F, K, AUXK = 1024, 1024, 65536, 32, 256
DEAD_STEPS = 200
NC, NS, L = 2, 16, 16
NW = NC * NS
RW = B // NW
CH = 16384
NCH = F // CH
GW = 16
NBINS = 2048
C1 = 256 + 16
C2 = 768 + 16
FS = F // NW
BK = B * K
BKC = 8192
FB = 2048

_mesh = plsc.VectorSubcoreMesh(core_axis_name="c", subcore_axis_name="s")


def _enc_body(xc_ref, w_ref, b_ref, lat_ref, sc_ref):
    acc = jnp.dot(xc_ref[...], w_ref[...], preferred_element_type=jnp.float32)
    lat_ref[...] = acc
    sc_ref[...] = acc + b_ref[...]


def _encode(xc, W_enc, latent_bias):
    return pl.pallas_call(
        _enc_body,
        grid=(F // FB,),
        in_specs=[pl.BlockSpec((B, D), lambda j: (0, 0)),
                  pl.BlockSpec((D, FB), lambda j: (0, j)),
                  pl.BlockSpec((1, FB), lambda j: (0, j))],
        out_specs=[pl.BlockSpec((B, FB), lambda j: (0, j)),
                   pl.BlockSpec((B, FB), lambda j: (0, j))],
        out_shape=[jax.ShapeDtypeStruct((B, F), jnp.float32),
                   jax.ShapeDtypeStruct((B, F), jnp.float32)],
    )(xc, W_enc, latent_bias.reshape(1, F))


def _zero_hist(hist):
    zero16 = jnp.zeros((L,), jnp.int32)

    def zb(i, c):
        hist[pl.ds(i * L, L)] = zero16
        return c

    lax.fori_loop(0, NBINS // L, zb, 0)


def _hist_threshold(hist, k):
    def cb(i, carry):
        srun, cntv = carry
        hv = hist[pl.ds((NBINS // L - 1 - i) * L, L)]
        sufv = lax.rev(plsc.cumsum(lax.rev(hv, (0,)), ), (0,)) + srun
        cntv = cntv + jnp.where(sufv >= k, 1, 0)
        return srun + jnp.sum(hv), cntv

    _, cntv = lax.fori_loop(0, NBINS // L, cb,
                            (jnp.int32(0), jnp.zeros((L,), jnp.int32)))
    return jnp.maximum(jnp.sum(cntv) - 2, 0)


def _bin_to_f32(t):
    return lax.bitcast_convert_type(jnp.broadcast_to(t, (L,)) << 20,
                                    jnp.float32)


_SC_PARAMS = pltpu.CompilerParams(needs_layout_passes=False)


@functools.partial(
    pl.kernel,
    compiler_params=_SC_PARAMS,
    out_type=(jax.ShapeDtypeStruct((B, C1), jnp.float32),
              jax.ShapeDtypeStruct((B, C1), jnp.int32)),
    mesh=_mesh,
    scratch_types=[
        pltpu.VMEM((CH,), jnp.float32),
        pltpu.VMEM((CH,), jnp.float32),
        pltpu.VMEM((NBINS,), jnp.int32),
        pltpu.VMEM((C1,), jnp.float32),
        pltpu.VMEM((C1,), jnp.int32),
        pltpu.SemaphoreType.DMA,
        pltpu.SemaphoreType.DMA,
    ],
)
def _topk_candidates(scores_hbm, cval_hbm, cidx_hbm,
                     chunk0, chunk1, hist, cval, cidx, sem0, sem1):
    wid = lax.axis_index("s") * NC + lax.axis_index("c")
    sems = (sem0, sem1)
    bufs = (chunk0, chunk1)
    one16 = jnp.ones((L,), jnp.int32)
    neg16 = jnp.full((L,), -3.0e38, jnp.float32)
    iota = lax.iota(jnp.int32, L)
    CAP = C1 - L

    def fill_neg(i, c):
        cval[pl.ds(i * L, L)] = neg16
        return c

    def do_row(ri, t):
        row = wid * RW + ri
        thr_g = _bin_to_f32(jnp.maximum(t - 1, 1))

        def issue(c):
            return pltpu.async_copy(
                scores_hbm.at[row, pl.ds(c * CH, CH)],
                bufs[c % 2], sems[c % 2])

        lax.fori_loop(0, C1 // L, fill_neg, 0)
        pend = issue(0)
        cntv = jnp.zeros((L,), jnp.int32)
        for c in range(NCH):
            nxt = issue(c + 1) if c + 1 < NCH else None
            pend.wait()
            ck = bufs[c % 2]

            def grp(g, cntv):
                base = g * (GW * L)
                vs = [ck[pl.ds(base + j * L, L)] for j in range(GW)]
                ms = [v >= thr_g for v in vs]
                om = ms[0]
                for j in range(1, GW):
                    om = jnp.logical_or(om, ms[j])
                any_s = jnp.max(plsc.all_reduce_population_count(om))

                def append(cntv):
                    for j in range(GW):
                        m = ms[j]
                        pos = plsc.cumsum(jnp.where(m, 1, 0)) - 1 + cntv
                        okm = jnp.logical_and(m, pos < CAP)
                        plsc.store_scatter(cval, [pos], vs[j], mask=okm)
                        plsc.store_scatter(
                            cidx, [pos],
                            iota + (c * CH + base + j * L), mask=okm)
                        cntv = jnp.minimum(
                            cntv + plsc.all_reduce_population_count(m), CAP)
                    return cntv

                return lax.cond(any_s > 0, append, lambda cv: cv, cntv)

            cntv = lax.fori_loop(0, CH // L // GW, grp, cntv)
            pend = nxt

        cnt = jnp.max(cntv)
        fail = jnp.logical_or(cnt < K, cnt >= CAP)

        def fallback(_):
            _zero_hist(hist)
            pend = issue(0)
            for c in range(NCH):
                nxt = issue(c + 1) if c + 1 < NCH else None
                pend.wait()
                ck = bufs[c % 2]

                def ph1(i, carry):
                    for j in range(8):
                        v = ck[pl.ds((i * 8 + j) * L, L)]
                        u = lax.bitcast_convert_type(v, jnp.int32)
                        bn = jnp.bitwise_and(u >> 20, NBINS - 1)
                        plsc.addupdate_scatter(hist, [bn], one16,
                                               mask=v > 0.0)
                    return carry

                lax.fori_loop(0, CH // L // 8, ph1, 0)
                pend = nxt

            te = jnp.maximum(_hist_threshold(hist, K), 1)
            thr = _bin_to_f32(te)
            lax.fori_loop(0, C1 // L, fill_neg, 0)
            pend = issue(0)
            cntv = jnp.zeros((L,), jnp.int32)
            for c in range(NCH):
                nxt = issue(c + 1) if c + 1 < NCH else None
                pend.wait()
                ck = bufs[c % 2]

                def ph3(i, cntv):
                    for j in range(8):
                        off = (i * 8 + j) * L
                        v = ck[pl.ds(off, L)]
                        m = jnp.logical_and(v >= thr, v > 0.0)
                        pos = plsc.cumsum(jnp.where(m, 1, 0)) - 1 + cntv
                        okm = jnp.logical_and(m, pos < CAP)
                        plsc.store_scatter(cval, [pos], v, mask=okm)
                        plsc.store_scatter(cidx, [pos],
                                           iota + (c * CH + off), mask=okm)
                        cntv = jnp.minimum(
                            cntv + plsc.all_reduce_population_count(m), CAP)
                    return cntv

                cntv = lax.fori_loop(0, CH // L // 8, ph3, cntv)
                pend = nxt
            return te

        def adapt(t):
            t = t + jnp.where(cnt > 3 * K, 1, 0) - jnp.where(cnt < 2 * K, 1, 0)
            return jnp.clip(t, 1, NBINS - 1)

        t = lax.cond(fail, fallback, adapt, t)
        pltpu.sync_copy(cval, cval_hbm.at[row])
        pltpu.sync_copy(cidx, cidx_hbm.at[row])
        return t

    lax.fori_loop(0, RW, do_row, jnp.int32(NBINS - 1))


@functools.partial(
    pl.kernel,
    compiler_params=_SC_PARAMS,
    out_type=jax.ShapeDtypeStruct((F,), jnp.int32),
    mesh=_mesh,
    scratch_types=[
        pltpu.VMEM((BKC,), jnp.int32),
        pltpu.VMEM((BKC,), jnp.float32),
        pltpu.VMEM((FS,), jnp.int32),
        pltpu.VMEM((FS,), jnp.int32),
    ],
)
def _stats(tidx_hbm, tval_hbm, last_hbm, stats_hbm,
           idxc, valc, counts, staging):
    wid = lax.axis_index("s") * NC + lax.axis_index("c")
    lo = wid * FS
    one16 = jnp.ones((L,), jnp.int32)
    zero16 = jnp.zeros((L,), jnp.int32)

    def zb(i, c):
        counts[pl.ds(i * L, L)] = zero16
        return c

    lax.fori_loop(0, FS // L, zb, 0)

    for c in range(BK // BKC):
        pltpu.sync_copy(tidx_hbm.at[pl.ds(c * BKC, BKC)], idxc)
        pltpu.sync_copy(tval_hbm.at[pl.ds(c * BKC, BKC)], valc)

        def sb(i, carry):
            for j in range(8):
                off = (i * 8 + j) * L
                iv = idxc[pl.ds(off, L)]
                vv = valc[pl.ds(off, L)]
                m = (iv >= lo) & (iv < lo + FS) & (vv > 0.001)
                li = jnp.bitwise_and(iv - lo, FS - 1)
                plsc.addupdate_scatter(counts, [li], one16, mask=m)
            return carry

        lax.fori_loop(0, BKC // L // 8, sb, 0)

    pltpu.sync_copy(last_hbm.at[pl.ds(lo, FS)], staging)

    def fin(i, c):
        cv = counts[pl.ds(i * L, L)]
        lv = staging[pl.ds(i * L, L)]
        staging[pl.ds(i * L, L)] = jnp.where(cv > 0, 1, lv + 1)
        return c

    lax.fori_loop(0, FS // L, fin, 0)
    pltpu.sync_copy(staging, stats_hbm.at[pl.ds(lo, FS)])


@functools.partial(
    pl.kernel,
    compiler_params=_SC_PARAMS,
    out_type=(jax.ShapeDtypeStruct((B, C2), jnp.float32),
              jax.ShapeDtypeStruct((B, C2), jnp.int32),
              jax.ShapeDtypeStruct((B, D), jnp.float32)),
    mesh=_mesh,
    scratch_types=[
        pltpu.VMEM((CH,), jnp.float32),
        pltpu.VMEM((CH,), jnp.float32),
        pltpu.VMEM((CH,), jnp.int32),
        pltpu.VMEM((CH,), jnp.int32),
        pltpu.VMEM((NBINS,), jnp.int32),
        pltpu.VMEM((C2,), jnp.float32),
        pltpu.VMEM((C2,), jnp.int32),
        pltpu.VMEM((K,), jnp.int32),
        pltpu.VMEM((K,), jnp.float32),
        pltpu.VMEM((K, D), jnp.float32),
        pltpu.VMEM((D,), jnp.float32),
        pltpu.VMEM((D,), jnp.float32),
        pltpu.VMEM_SHARED((F,), jnp.int32),
        pltpu.SemaphoreType.DMA,
        pltpu.SemaphoreType.DMA,
        pltpu.SemaphoreType.DMA,
        pltpu.SemaphoreType.DMA,
        pltpu.SemaphoreType.DMA,
    ],
)
def _aux_decode(lat_hbm, stats_hbm, tidx_hbm, tval_hbm, wdec_hbm, bias_hbm,
                aval_hbm, aidx_hbm, rec_hbm,
                chunk0, chunk1, schunk0, schunk1, hist, aval, aidx,
                idxrow, valrow, rows,
                accbuf, biasbuf, stats_sh, sem0, sem1, sem2, sem3, semg):
    bufs = (chunk0, chunk1)
    sbufs = (schunk0, schunk1)
    ssems = (sem2, sem3)
    sid = lax.axis_index("s")
    cid = lax.axis_index("c")
    wid = sid * NC + cid

    @pl.when(sid == 0)
    def _():
        pltpu.sync_copy(stats_hbm, stats_sh)

    pltpu.sync_copy(bias_hbm, biasbuf)
    plsc.subcore_barrier()

    sems = (sem0, sem1)
    one16 = jnp.ones((L,), jnp.int32)
    neg16 = jnp.full((L,), -3.0e38, jnp.float32)
    iota = lax.iota(jnp.int32, L)
    CAP = C2 - L

    def fill_neg(i, c):
        aval[pl.ds(i * L, L)] = neg16
        return c

    def do_row(ri, t):
        row = wid * RW + ri
        thr_g = _bin_to_f32(jnp.maximum(t - 1, 1))

        pltpu.sync_copy(tidx_hbm.at[row], idxrow)
        pltpu.sync_copy(tval_hbm.at[row], valrow)
        gather = pltpu.async_copy(wdec_hbm.at[idxrow], rows, semg)

        def issue(c):
            return (pltpu.async_copy(
                        lat_hbm.at[row, pl.ds(c * CH, CH)],
                        bufs[c % 2], sems[c % 2]),
                    pltpu.async_copy(
                        stats_sh.at[pl.ds(c * CH, CH)],
                        sbufs[c % 2], ssems[c % 2]))

        def wait(p):
            p[0].wait()
            p[1].wait()

        lax.fori_loop(0, C2 // L, fill_neg, 0)
        pend = issue(0)
        cntv = jnp.zeros((L,), jnp.int32)
        for c in range(NCH):
            nxt = issue(c + 1) if c + 1 < NCH else None
            wait(pend)
            ck = bufs[c % 2]
            sk = sbufs[c % 2]

            def grp(g, cntv):
                base = g * (GW * L)
                vs = [ck[pl.ds(base + j * L, L)] for j in range(GW)]
                ss = [sk[pl.ds(base + j * L, L)] for j in range(GW)]
                ms = [jnp.logical_and(s > DEAD_STEPS, v >= thr_g)
                      for v, s in zip(vs, ss)]
                om = ms[0]
                for j in range(1, GW):
                    om = jnp.logical_or(om, ms[j])
                any_s = jnp.max(plsc.all_reduce_population_count(om))

                def append(cntv):
                    for j in range(GW):
                        m = ms[j]
                        pos = plsc.cumsum(jnp.where(m, 1, 0)) - 1 + cntv
                        okm = jnp.logical_and(m, pos < CAP)
                        plsc.store_scatter(aval, [pos], vs[j], mask=okm)
                        plsc.store_scatter(
                            aidx, [pos],
                            iota + (c * CH + base + j * L), mask=okm)
                        cntv = jnp.minimum(
                            cntv + plsc.all_reduce_population_count(m), CAP)
                    return cntv

                return lax.cond(any_s > 0, append, lambda cv: cv, cntv)

            cntv = lax.fori_loop(0, CH // L // GW, grp, cntv)
            pend = nxt

        cnt = jnp.max(cntv)
        fail = jnp.logical_or(cnt < AUXK, cnt >= CAP)

        def fallback(_):
            _zero_hist(hist)
            pend = issue(0)
            for c in range(NCH):
                nxt = issue(c + 1) if c + 1 < NCH else None
                wait(pend)
                ck = bufs[c % 2]
                sk = sbufs[c % 2]

                def ph1(i, carry):
                    for j in range(8):
                        off = (i * 8 + j) * L
                        v = ck[pl.ds(off, L)]
                        sv = sk[pl.ds(off, L)]
                        m = (sv > DEAD_STEPS) & (v > 0.0)
                        u = lax.bitcast_convert_type(v, jnp.int32)
                        bn = jnp.bitwise_and(u >> 20, NBINS - 1)
                        plsc.addupdate_scatter(hist, [bn], one16, mask=m)
                    return carry

                lax.fori_loop(0, CH // L // 8, ph1, 0)
                pend = nxt

            te = jnp.maximum(_hist_threshold(hist, AUXK), 1)
            thr = _bin_to_f32(te)
            lax.fori_loop(0, C2 // L, fill_neg, 0)
            pend = issue(0)
            cntv = jnp.zeros((L,), jnp.int32)
            for c in range(NCH):
                nxt = issue(c + 1) if c + 1 < NCH else None
                wait(pend)
                ck = bufs[c % 2]
                sk = sbufs[c % 2]

                def ph3(i, cntv):
                    for j in range(8):
                        off = (i * 8 + j) * L
                        v = ck[pl.ds(off, L)]
                        sv = sk[pl.ds(off, L)]
                        m = (sv > DEAD_STEPS) & (v >= thr)
                        pos = plsc.cumsum(jnp.where(m, 1, 0)) - 1 + cntv
                        okm = jnp.logical_and(m, pos < CAP)
                        plsc.store_scatter(aval, [pos], v, mask=okm)
                        plsc.store_scatter(aidx, [pos],
                                           iota + (c * CH + off), mask=okm)
                        cntv = jnp.minimum(
                            cntv + plsc.all_reduce_population_count(m), CAP)
                    return cntv

                cntv = lax.fori_loop(0, CH // L // 8, ph3, cntv)
                pend = nxt
            return te

        def adapt(t):
            t = (t + jnp.where(cnt > AUXK + AUXK // 2, 1, 0)
                 - jnp.where(cnt < AUXK + AUXK // 4, 1, 0))
            return jnp.clip(t, 1, NBINS - 1)

        t = lax.cond(fail, fallback, adapt, t)
        pltpu.sync_copy(aval, aval_hbm.at[row])
        pltpu.sync_copy(aidx, aidx_hbm.at[row])

        gather.wait()
        vv0 = valrow[pl.ds(0, L)]
        vv1 = valrow[pl.ds(L, L)]
        vals_sc = [jnp.sum(jnp.where(iota == (k % L), vv0 if k < L else vv1,
                                     0.0))
                   for k in range(K)]

        def dec(dc, carry):
            accs = [biasbuf[pl.ds(dc * L, L)]] + [jnp.zeros((L,), jnp.float32)
                                                  for _ in range(3)]
            for k in range(K):
                accs[k % 4] = accs[k % 4] + vals_sc[k] * rows[k, pl.ds(dc * L, L)]
            accbuf[pl.ds(dc * L, L)] = ((accs[0] + accs[1])
                                        + (accs[2] + accs[3]))
            return carry

        lax.fori_loop(0, D // L, dec, 0)
        pltpu.sync_copy(accbuf, rec_hbm.at[row])
        return t

    lax.fori_loop(0, RW, do_row, jnp.int32(NBINS - 1))


def kernel(x, pre_bias, W_enc, latent_bias, W_dec, stats_last_nonzero):
    xc = x - pre_bias
    latents, scores = _encode(xc, W_enc, latent_bias)

    cval, cidx = _topk_candidates(scores)
    pv, pp = lax.top_k(cval, K)
    topk_idxs = jnp.take_along_axis(cidx, pp, axis=1)
    topk_vals = jnp.maximum(pv, 0.0)

    stats = _stats(topk_idxs.reshape(-1), pv.reshape(-1), stats_last_nonzero)

    aval, aidx, recons = _aux_decode(latents, stats, topk_idxs, topk_vals,
                                     W_dec, pre_bias)
    av, ap = lax.top_k(aval, AUXK)
    auxk_idxs = jnp.take_along_axis(aidx, ap, axis=1)
    auxk_vals = jnp.maximum(av, 0.0)

    return recons, topk_idxs, topk_vals, auxk_idxs, auxk_vals, stats


# speedup vs baseline: 1.4495x; 1.4495x over previous
import functools

import jax
import jax.numpy as jnp
from jax import lax
from jax.experimental import pallas as pl
from jax.experimental.pallas import tpu as pltpu
from jax.experimental.pallas import tpu_sc as plsc

B, D, F, K, AUXK = 1024, 1024, 65536, 32, 256
DEAD_STEPS = 200
NC, NS, L = 2, 16, 16
NW = NC * NS
RW = B // NW
CH = 16384
NCH = F // CH
GW = 16
NBINS = 2048
C1 = 512 + 16
C2 = 1024 + 16
FS = F // NW
BK = B * K
BKC = 8192
FB = 2048

_mesh = plsc.VectorSubcoreMesh(core_axis_name="c", subcore_axis_name="s")


def _enc_body(xc_ref, w_ref, b_ref, lat_ref, sc_ref):
    acc = jnp.dot(xc_ref[...], w_ref[...], preferred_element_type=jnp.float32)
    lat_ref[...] = acc
    sc_ref[...] = acc + b_ref[...]


def _encode(xc, W_enc, latent_bias):
    return pl.pallas_call(
        _enc_body,
        grid=(F // FB,),
        in_specs=[pl.BlockSpec((B, D), lambda j: (0, 0)),
                  pl.BlockSpec((D, FB), lambda j: (0, j)),
                  pl.BlockSpec((1, FB), lambda j: (0, j))],
        out_specs=[pl.BlockSpec((B, FB), lambda j: (0, j)),
                   pl.BlockSpec((B, FB), lambda j: (0, j))],
        out_shape=[jax.ShapeDtypeStruct((B, F), jnp.float32),
                   jax.ShapeDtypeStruct((B, F), jnp.float32)],
    )(xc, W_enc, latent_bias.reshape(1, F))


def _zero_hist(hist):
    zero16 = jnp.zeros((L,), jnp.int32)

    def zb(i, c):
        hist[pl.ds(i * L, L)] = zero16
        return c

    lax.fori_loop(0, NBINS // L, zb, 0)


def _hist_threshold(hist, k):
    def cb(i, carry):
        srun, cntv = carry
        hv = hist[pl.ds((NBINS // L - 1 - i) * L, L)]
        sufv = lax.rev(plsc.cumsum(lax.rev(hv, (0,)), ), (0,)) + srun
        cntv = cntv + jnp.where(sufv >= k, 1, 0)
        return srun + jnp.sum(hv), cntv

    _, cntv = lax.fori_loop(0, NBINS // L, cb,
                            (jnp.int32(0), jnp.zeros((L,), jnp.int32)))
    return jnp.maximum(jnp.sum(cntv) - 2, 0)


def _bin_to_f32(t):
    return lax.bitcast_convert_type(jnp.broadcast_to(t, (L,)) << 20,
                                    jnp.float32)


_SC_PARAMS = pltpu.CompilerParams(needs_layout_passes=False)


@functools.partial(
    pl.kernel,
    compiler_params=_SC_PARAMS,
    out_type=(jax.ShapeDtypeStruct((B, C1), jnp.float32),
              jax.ShapeDtypeStruct((B, C1), jnp.int32)),
    mesh=_mesh,
    scratch_types=[
        pltpu.VMEM((CH,), jnp.float32),
        pltpu.VMEM((CH,), jnp.float32),
        pltpu.VMEM((NBINS,), jnp.int32),
        pltpu.VMEM((C1,), jnp.float32),
        pltpu.VMEM((C1,), jnp.int32),
        pltpu.SemaphoreType.DMA,
        pltpu.SemaphoreType.DMA,
    ],
)
def _topk_candidates(scores_hbm, cval_hbm, cidx_hbm,
                     chunk0, chunk1, hist, cval, cidx, sem0, sem1):
    wid = lax.axis_index("s") * NC + lax.axis_index("c")
    sems = (sem0, sem1)
    bufs = (chunk0, chunk1)
    one16 = jnp.ones((L,), jnp.int32)
    neg16 = jnp.full((L,), -3.0e38, jnp.float32)
    iota = lax.iota(jnp.int32, L)
    CAP = C1 - L

    def fill_neg(i, c):
        cval[pl.ds(i * L, L)] = neg16
        return c

    def do_row(ri, t):
        row = wid * RW + ri
        thr_g = _bin_to_f32(t)

        def issue(c):
            return pltpu.async_copy(
                scores_hbm.at[row, pl.ds(c * CH, CH)],
                bufs[c % 2], sems[c % 2])

        lax.fori_loop(0, C1 // L, fill_neg, 0)
        pend = issue(0)
        cntv = jnp.zeros((L,), jnp.int32)
        for c in range(NCH):
            nxt = issue(c + 1) if c + 1 < NCH else None
            pend.wait()
            ck = bufs[c % 2]

            def grp(g, cntv):
                base = g * (GW * L)
                vs = [ck[pl.ds(base + j * L, L)] for j in range(GW)]
                ms = [v >= thr_g for v in vs]
                om = ms[0]
                for j in range(1, GW):
                    om = jnp.logical_or(om, ms[j])
                any_s = jnp.max(plsc.all_reduce_population_count(om))

                def append(cntv):
                    for j in range(GW):
                        m = ms[j]
                        pos = plsc.cumsum(jnp.where(m, 1, 0)) - 1 + cntv
                        okm = jnp.logical_and(m, pos < CAP)
                        plsc.store_scatter(cval, [pos], vs[j], mask=okm)
                        plsc.store_scatter(
                            cidx, [pos],
                            iota + (c * CH + base + j * L), mask=okm)
                        cntv = jnp.minimum(
                            cntv + plsc.all_reduce_population_count(m), CAP)
                    return cntv

                return lax.cond(any_s > 0, append, lambda cv: cv, cntv)

            cntv = lax.fori_loop(0, CH // L // GW, grp, cntv)
            pend = nxt

        cnt = jnp.max(cntv)
        fail = jnp.logical_or(cnt < K, cnt >= CAP)

        def fallback(_):
            _zero_hist(hist)
            pend = issue(0)
            for c in range(NCH):
                nxt = issue(c + 1) if c + 1 < NCH else None
                pend.wait()
                ck = bufs[c % 2]

                def ph1(i, carry):
                    for j in range(8):
                        v = ck[pl.ds((i * 8 + j) * L, L)]
                        u = lax.bitcast_convert_type(v, jnp.int32)
                        bn = jnp.bitwise_and(u >> 20, NBINS - 1)
                        plsc.addupdate_scatter(hist, [bn], one16,
                                               mask=v > 0.0)
                    return carry

                lax.fori_loop(0, CH // L // 8, ph1, 0)
                pend = nxt

            te = jnp.maximum(_hist_threshold(hist, K), 1)
            thr = _bin_to_f32(te)
            lax.fori_loop(0, C1 // L, fill_neg, 0)
            pend = issue(0)
            cntv = jnp.zeros((L,), jnp.int32)
            for c in range(NCH):
                nxt = issue(c + 1) if c + 1 < NCH else None
                pend.wait()
                ck = bufs[c % 2]

                def ph3(i, cntv):
                    for j in range(8):
                        off = (i * 8 + j) * L
                        v = ck[pl.ds(off, L)]
                        m = jnp.logical_and(v >= thr, v > 0.0)
                        pos = plsc.cumsum(jnp.where(m, 1, 0)) - 1 + cntv
                        okm = jnp.logical_and(m, pos < CAP)
                        plsc.store_scatter(cval, [pos], v, mask=okm)
                        plsc.store_scatter(cidx, [pos],
                                           iota + (c * CH + off), mask=okm)
                        cntv = jnp.minimum(
                            cntv + plsc.all_reduce_population_count(m), CAP)
                    return cntv

                cntv = lax.fori_loop(0, CH // L // 8, ph3, cntv)
                pend = nxt
            return te

        def adapt(t):
            t = t + jnp.where(cnt > 3 * K, 1, 0) - jnp.where(cnt < 2 * K, 1, 0)
            return jnp.clip(t, 1, NBINS - 1)

        t = lax.cond(fail, fallback, adapt, t)
        pltpu.sync_copy(cval, cval_hbm.at[row])
        pltpu.sync_copy(cidx, cidx_hbm.at[row])
        return t

    lax.fori_loop(0, RW, do_row, jnp.int32(NBINS - 1))


@functools.partial(
    pl.kernel,
    compiler_params=_SC_PARAMS,
    out_type=jax.ShapeDtypeStruct((F,), jnp.int32),
    mesh=_mesh,
    scratch_types=[
        pltpu.VMEM((BKC,), jnp.int32),
        pltpu.VMEM((BKC,), jnp.float32),
        pltpu.VMEM((FS,), jnp.int32),
        pltpu.VMEM((FS,), jnp.int32),
    ],
)
def _stats(tidx_hbm, tval_hbm, last_hbm, stats_hbm,
           idxc, valc, counts, staging):
    wid = lax.axis_index("s") * NC + lax.axis_index("c")
    lo = wid * FS
    one16 = jnp.ones((L,), jnp.int32)
    zero16 = jnp.zeros((L,), jnp.int32)

    def zb(i, c):
        counts[pl.ds(i * L, L)] = zero16
        return c

    lax.fori_loop(0, FS // L, zb, 0)

    for c in range(BK // BKC):
        pltpu.sync_copy(tidx_hbm.at[pl.ds(c * BKC, BKC)], idxc)
        pltpu.sync_copy(tval_hbm.at[pl.ds(c * BKC, BKC)], valc)

        def sb(i, carry):
            for j in range(8):
                off = (i * 8 + j) * L
                iv = idxc[pl.ds(off, L)]
                vv = valc[pl.ds(off, L)]
                m = (iv >= lo) & (iv < lo + FS) & (vv > 0.001)
                li = jnp.bitwise_and(iv - lo, FS - 1)
                plsc.addupdate_scatter(counts, [li], one16, mask=m)
            return carry

        lax.fori_loop(0, BKC // L // 8, sb, 0)

    pltpu.sync_copy(last_hbm.at[pl.ds(lo, FS)], staging)

    def fin(i, c):
        cv = counts[pl.ds(i * L, L)]
        lv = staging[pl.ds(i * L, L)]
        staging[pl.ds(i * L, L)] = jnp.where(cv > 0, 1, lv + 1)
        return c

    lax.fori_loop(0, FS // L, fin, 0)
    pltpu.sync_copy(staging, stats_hbm.at[pl.ds(lo, FS)])


@functools.partial(
    pl.kernel,
    compiler_params=_SC_PARAMS,
    out_type=(jax.ShapeDtypeStruct((B, C2), jnp.float32),
              jax.ShapeDtypeStruct((B, C2), jnp.int32),
              jax.ShapeDtypeStruct((B, D), jnp.float32)),
    mesh=_mesh,
    scratch_types=[
        pltpu.VMEM((CH,), jnp.float32),
        pltpu.VMEM((CH,), jnp.float32),
        pltpu.VMEM((CH,), jnp.int32),
        pltpu.VMEM((CH,), jnp.int32),
        pltpu.VMEM((NBINS,), jnp.int32),
        pltpu.VMEM((C2,), jnp.float32),
        pltpu.VMEM((C2,), jnp.int32),
        pltpu.VMEM((K,), jnp.int32),
        pltpu.VMEM((K,), jnp.float32),
        pltpu.VMEM((K, D), jnp.float32),
        pltpu.VMEM((D,), jnp.float32),
        pltpu.VMEM((D,), jnp.float32),
        pltpu.VMEM_SHARED((F,), jnp.int32),
        pltpu.SemaphoreType.DMA,
        pltpu.SemaphoreType.DMA,
        pltpu.SemaphoreType.DMA,
        pltpu.SemaphoreType.DMA,
        pltpu.SemaphoreType.DMA,
    ],
)
def _aux_decode(lat_hbm, stats_hbm, tidx_hbm, tval_hbm, wdec_hbm, bias_hbm,
                aval_hbm, aidx_hbm, rec_hbm,
                chunk0, chunk1, schunk0, schunk1, hist, aval, aidx,
                idxrow, valrow, rows,
                accbuf, biasbuf, stats_sh, sem0, sem1, sem2, sem3, semg):
    bufs = (chunk0, chunk1)
    sbufs = (schunk0, schunk1)
    ssems = (sem2, sem3)
    sid = lax.axis_index("s")
    cid = lax.axis_index("c")
    wid = sid * NC + cid

    @pl.when(sid == 0)
    def _():
        pltpu.sync_copy(stats_hbm, stats_sh)

    pltpu.sync_copy(bias_hbm, biasbuf)
    plsc.subcore_barrier()

    sems = (sem0, sem1)
    one16 = jnp.ones((L,), jnp.int32)
    neg16 = jnp.full((L,), -3.0e38, jnp.float32)
    iota = lax.iota(jnp.int32, L)
    CAP = C2 - L

    def fill_neg(i, c):
        aval[pl.ds(i * L, L)] = neg16
        return c

    def do_row(ri, t):
        row = wid * RW + ri
        thr_g = _bin_to_f32(t)

        pltpu.sync_copy(tidx_hbm.at[row], idxrow)
        pltpu.sync_copy(tval_hbm.at[row], valrow)
        gather = pltpu.async_copy(wdec_hbm.at[idxrow], rows, semg)

        def issue(c):
            return (pltpu.async_copy(
                        lat_hbm.at[row, pl.ds(c * CH, CH)],
                        bufs[c % 2], sems[c % 2]),
                    pltpu.async_copy(
                        stats_sh.at[pl.ds(c * CH, CH)],
                        sbufs[c % 2], ssems[c % 2]))

        def wait(p):
            p[0].wait()
            p[1].wait()

        lax.fori_loop(0, C2 // L, fill_neg, 0)
        pend = issue(0)
        cntv = jnp.zeros((L,), jnp.int32)
        for c in range(NCH):
            nxt = issue(c + 1) if c + 1 < NCH else None
            wait(pend)
            ck = bufs[c % 2]
            sk = sbufs[c % 2]

            def grp(g, cntv):
                base = g * (GW * L)
                vs = [ck[pl.ds(base + j * L, L)] for j in range(GW)]
                ss = [sk[pl.ds(base + j * L, L)] for j in range(GW)]
                ms = [jnp.logical_and(s > DEAD_STEPS, v >= thr_g)
                      for v, s in zip(vs, ss)]
                om = ms[0]
                for j in range(1, GW):
                    om = jnp.logical_or(om, ms[j])
                any_s = jnp.max(plsc.all_reduce_population_count(om))

                def append(cntv):
                    for j in range(GW):
                        m = ms[j]
                        pos = plsc.cumsum(jnp.where(m, 1, 0)) - 1 + cntv
                        okm = jnp.logical_and(m, pos < CAP)
                        plsc.store_scatter(aval, [pos], vs[j], mask=okm)
                        plsc.store_scatter(
                            aidx, [pos],
                            iota + (c * CH + base + j * L), mask=okm)
                        cntv = jnp.minimum(
                            cntv + plsc.all_reduce_population_count(m), CAP)
                    return cntv

                return lax.cond(any_s > 0, append, lambda cv: cv, cntv)

            cntv = lax.fori_loop(0, CH // L // GW, grp, cntv)
            pend = nxt

        cnt = jnp.max(cntv)
        fail = jnp.logical_or(cnt < AUXK, cnt >= CAP)

        def fallback(_):
            _zero_hist(hist)
            pend = issue(0)
            for c in range(NCH):
                nxt = issue(c + 1) if c + 1 < NCH else None
                wait(pend)
                ck = bufs[c % 2]
                sk = sbufs[c % 2]

                def ph1(i, carry):
                    for j in range(8):
                        off = (i * 8 + j) * L
                        v = ck[pl.ds(off, L)]
                        sv = sk[pl.ds(off, L)]
                        m = (sv > DEAD_STEPS) & (v > 0.0)
                        u = lax.bitcast_convert_type(v, jnp.int32)
                        bn = jnp.bitwise_and(u >> 20, NBINS - 1)
                        plsc.addupdate_scatter(hist, [bn], one16, mask=m)
                    return carry

                lax.fori_loop(0, CH // L // 8, ph1, 0)
                pend = nxt

            te = jnp.maximum(_hist_threshold(hist, AUXK), 1)
            thr = _bin_to_f32(te)
            lax.fori_loop(0, C2 // L, fill_neg, 0)
            pend = issue(0)
            cntv = jnp.zeros((L,), jnp.int32)
            for c in range(NCH):
                nxt = issue(c + 1) if c + 1 < NCH else None
                wait(pend)
                ck = bufs[c % 2]
                sk = sbufs[c % 2]

                def ph3(i, cntv):
                    for j in range(8):
                        off = (i * 8 + j) * L
                        v = ck[pl.ds(off, L)]
                        sv = sk[pl.ds(off, L)]
                        m = (sv > DEAD_STEPS) & (v >= thr)
                        pos = plsc.cumsum(jnp.where(m, 1, 0)) - 1 + cntv
                        okm = jnp.logical_and(m, pos < CAP)
                        plsc.store_scatter(aval, [pos], v, mask=okm)
                        plsc.store_scatter(aidx, [pos],
                                           iota + (c * CH + off), mask=okm)
                        cntv = jnp.minimum(
                            cntv + plsc.all_reduce_population_count(m), CAP)
                    return cntv

                cntv = lax.fori_loop(0, CH // L // 8, ph3, cntv)
                pend = nxt
            return te

        def adapt(t):
            t = (t + jnp.where(cnt > AUXK + AUXK // 2, 1, 0)
                 - jnp.where(cnt < AUXK + AUXK // 4, 1, 0))
            return jnp.clip(t, 1, NBINS - 1)

        t = lax.cond(fail, fallback, adapt, t)
        pltpu.sync_copy(aval, aval_hbm.at[row])
        pltpu.sync_copy(aidx, aidx_hbm.at[row])

        gather.wait()
        vv0 = valrow[pl.ds(0, L)]
        vv1 = valrow[pl.ds(L, L)]
        vals_sc = [jnp.sum(jnp.where(iota == (k % L), vv0 if k < L else vv1,
                                     0.0))
                   for k in range(K)]

        def dec(dc, carry):
            accs = [biasbuf[pl.ds(dc * L, L)]] + [jnp.zeros((L,), jnp.float32)
                                                  for _ in range(3)]
            for k in range(K):
                accs[k % 4] = accs[k % 4] + vals_sc[k] * rows[k, pl.ds(dc * L, L)]
            accbuf[pl.ds(dc * L, L)] = ((accs[0] + accs[1])
                                        + (accs[2] + accs[3]))
            return carry

        lax.fori_loop(0, D // L, dec, 0)
        pltpu.sync_copy(accbuf, rec_hbm.at[row])
        return t

    lax.fori_loop(0, RW, do_row, jnp.int32(NBINS - 1))


def kernel(x, pre_bias, W_enc, latent_bias, W_dec, stats_last_nonzero):
    xc = x - pre_bias
    latents, scores = _encode(xc, W_enc, latent_bias)

    cval, cidx = _topk_candidates(scores)
    pv, pp = lax.top_k(cval, K)
    topk_idxs = jnp.take_along_axis(cidx, pp, axis=1)
    topk_vals = jnp.maximum(pv, 0.0)

    stats = _stats(topk_idxs.reshape(-1), pv.reshape(-1), stats_last_nonzero)

    aval, aidx, recons = _aux_decode(latents, stats, topk_idxs, topk_vals,
                                     W_dec, pre_bias)
    av, ap = lax.top_k(aval, AUXK)
    auxk_idxs = jnp.take_along_axis(aidx, ap, axis=1)
    auxk_vals = jnp.maximum(av, 0.0)

    return recons, topk_idxs, topk_vals, auxk_idxs, auxk_vals, stats
